# Initial kernel scaffold; baseline (speedup 1.0000x reference)
import jax
import jax.numpy as jnp
from jax.experimental import pallas as pl


def kernel(x, edge_index, edge_attr, batch, We0, be0, W10, b10, W20, b20, eps0, We1, be1, W11, b11, W21, b21, eps1):
    raise NotImplementedError("write your pallas kernel here")


# speedup vs baseline: 2.5821x; 2.5821x over previous
import functools

import jax
import jax.numpy as jnp
from jax import lax
from jax.experimental import pallas as pl
from jax.experimental.pallas import tpu as pltpu
from jax.experimental.pallas import tpu_sc as plsc

N = 10000
E = 320000
D = 128
DE = 16
G = 64

NC = 2
NS = 16
L = 16
NW = NC * NS
EPT = E // NW
CHUNK = 80
NCHUNK = EPT // CHUNK
NPAD = 10240
ROWS_PT = NPAD // NS


def _edge_attr_proj_kernel(attr_ref, w_ref, b_ref, out0_ref, out1_ref):
    ea = jnp.dot(attr_ref[...], w_ref[...], preferred_element_type=jnp.float32)
    ea = ea + b_ref[...]
    out0_ref[...] = ea[:, :D]
    out1_ref[...] = ea[:, D:]


def _edge_attr_proj(edge_attr, Wcat, bcat):
    BE = 8000
    grid = E // BE
    return pl.pallas_call(
        _edge_attr_proj_kernel,
        grid=(grid,),
        in_specs=[
            pl.BlockSpec((BE, DE), lambda i: (i, 0)),
            pl.BlockSpec((DE, 2 * D), lambda i: (0, 0)),
            pl.BlockSpec((1, 2 * D), lambda i: (0, 0)),
        ],
        out_specs=[
            pl.BlockSpec((BE, D), lambda i: (i, 0)),
            pl.BlockSpec((BE, D), lambda i: (i, 0)),
        ],
        out_shape=[
            jax.ShapeDtypeStruct((E, D), jnp.float32),
            jax.ShapeDtypeStruct((E, D), jnp.float32),
        ],
    )(edge_attr, Wcat, bcat)


def _edge_phase_body(x_hbm, ea_hbm, src_hbm, dst_hbm, out_hbm,
                     sidx, didx, eabuf, rows, aggr_sh, sem):
    c = lax.axis_index("c")
    s = lax.axis_index("s")
    wid = s * NC + c

    def zfill(i, _):
        for j in range(D // L):
            rows[i, pl.ds(j * L, L)] = jnp.zeros((L,), jnp.float32)
        return 0
    lax.fori_loop(0, CHUNK, zfill, 0)
    for j in range(ROWS_PT // CHUNK):
        pltpu.sync_copy(rows, aggr_sh.at[pl.ds(s * ROWS_PT + j * CHUNK, CHUNK)])
    plsc.subcore_barrier()

    def chunk_body(i, _):
        base = wid * EPT + i * CHUNK
        pltpu.sync_copy(src_hbm.at[pl.ds(base, CHUNK)], sidx)
        pltpu.sync_copy(dst_hbm.at[pl.ds(base, CHUNK)], didx)
        pltpu.sync_copy(ea_hbm.at[pl.ds(base, CHUNK)], eabuf)
        pltpu.async_copy(x_hbm.at[sidx], rows, sem).wait()

        def row_body(r, _):
            for j in range(D // L):
                sl = pl.ds(j * L, L)
                rows[r, sl] = jnp.maximum(rows[r, sl] + eabuf[r, sl], 0.0)
            return 0
        lax.fori_loop(0, CHUNK, row_body, 0)
        pltpu.sync_copy(rows, aggr_sh.at[didx], add=True)
        return 0
    lax.fori_loop(0, NCHUNK, chunk_body, 0)

    plsc.subcore_barrier()
    rs = s * ROWS_PT
    pltpu.sync_copy(aggr_sh.at[pl.ds(rs, ROWS_PT)],
                    out_hbm.at[c].at[pl.ds(rs, ROWS_PT)])


def _edge_phase(x, ea, src, dst):
    mesh = plsc.VectorSubcoreMesh(core_axis_name="c", subcore_axis_name="s")
    f = pl.kernel(
        _edge_phase_body,
        out_type=jax.ShapeDtypeStruct((NC, NPAD, D), jnp.float32),
        mesh=mesh,
        scratch_types=[
            pltpu.VMEM((CHUNK,), jnp.int32),
            pltpu.VMEM((CHUNK,), jnp.int32),
            pltpu.VMEM((CHUNK, D), jnp.float32),
            pltpu.VMEM((CHUNK, D), jnp.float32),
            pltpu.VMEM_SHARED((NPAD, D), jnp.float32),
            pltpu.SemaphoreType.DMA,
        ],
    )
    return f(x, ea, src, dst)


def _mlp_kernel(x_ref, p_ref, w1_ref, b1_ref, w2_ref, b2_ref, scale_ref, out_ref):
    z = scale_ref[0, 0] * x_ref[...] + p_ref[0] + p_ref[1]
    h = jnp.maximum(jnp.dot(z, w1_ref[...], preferred_element_type=jnp.float32)
                    + b1_ref[...], 0.0)
    out_ref[...] = jnp.dot(h, w2_ref[...], preferred_element_type=jnp.float32) + b2_ref[...]


def _mlp(x, parts, W1, b1, W2, b2, scale):
    BN = 1000
    grid = N // BN
    return pl.pallas_call(
        _mlp_kernel,
        grid=(grid,),
        in_specs=[
            pl.BlockSpec((BN, D), lambda i: (i, 0)),
            pl.BlockSpec((NC, BN, D), lambda i: (0, i, 0)),
            pl.BlockSpec((D, D), lambda i: (0, 0)),
            pl.BlockSpec((1, D), lambda i: (0, 0)),
            pl.BlockSpec((D, D), lambda i: (0, 0)),
            pl.BlockSpec((1, D), lambda i: (0, 0)),
            pl.BlockSpec(memory_space=pltpu.SMEM),
        ],
        out_specs=pl.BlockSpec((BN, D), lambda i: (i, 0)),
        out_shape=jax.ShapeDtypeStruct((N, D), jnp.float32),
    )(x, parts, W1, b1, W2, b2, scale)


def _mlp_pool_kernel(h1_ref, p_ref, w1_ref, b1_ref, w2_ref, b2_ref,
                     scale_ref, batch_ref, out_ref, sums, cnts):
    i = pl.program_id(0)
    n = pl.num_programs(0)

    @pl.when(i == 0)
    def _():
        sums[...] = jnp.zeros_like(sums)
        cnts[...] = jnp.zeros_like(cnts)

    z = scale_ref[0, 0] * h1_ref[...] + p_ref[0] + p_ref[1]
    m = jnp.maximum(jnp.dot(z, w1_ref[...], preferred_element_type=jnp.float32)
                    + b1_ref[...], 0.0)
    h = h1_ref[...] + jnp.dot(m, w2_ref[...], preferred_element_type=jnp.float32) + b2_ref[...]

    onehot = (lax.broadcasted_iota(jnp.int32, (G, h.shape[0]), 0)
              == batch_ref[0]).astype(jnp.float32)
    sums[...] += jnp.dot(onehot, h, preferred_element_type=jnp.float32)
    cnts[...] += jnp.sum(onehot, axis=1, keepdims=True)

    @pl.when(i == n - 1)
    def _():
        out_ref[...] = sums[...] / jnp.maximum(cnts[...], 1.0)


def _mlp_pool(h1, parts, W1, b1, W2, b2, scale, batch2d):
    BN = 1000
    grid = N // BN
    return pl.pallas_call(
        _mlp_pool_kernel,
        grid=(grid,),
        in_specs=[
            pl.BlockSpec((BN, D), lambda i: (i, 0)),
            pl.BlockSpec((NC, BN, D), lambda i: (0, i, 0)),
            pl.BlockSpec((D, D), lambda i: (0, 0)),
            pl.BlockSpec((1, D), lambda i: (0, 0)),
            pl.BlockSpec((D, D), lambda i: (0, 0)),
            pl.BlockSpec((1, D), lambda i: (0, 0)),
            pl.BlockSpec(memory_space=pltpu.SMEM),
            pl.BlockSpec((1, 1, BN), lambda i: (i, 0, 0)),
        ],
        out_specs=pl.BlockSpec((G, D), lambda i: (0, 0)),
        out_shape=jax.ShapeDtypeStruct((G, D), jnp.float32),
        scratch_shapes=[
            pltpu.VMEM((G, D), jnp.float32),
            pltpu.VMEM((G, 1), jnp.float32),
        ],
        compiler_params=pltpu.CompilerParams(
            dimension_semantics=("arbitrary",),
        ),
    )(h1, parts, W1, b1, W2, b2, scale, batch2d)


def kernel(x, edge_index, edge_attr, batch,
           We0, be0, W10, b10, W20, b20, eps0,
           We1, be1, W11, b11, W21, b21, eps1):
    src = edge_index[0]
    dst = edge_index[1]
    Wcat = jnp.concatenate([We0, We1], axis=1)
    bcat = jnp.concatenate([be0, be1])[None, :]
    ea0, ea1 = _edge_attr_proj(edge_attr, Wcat, bcat)

    scale0 = (1.0 + eps0).reshape(1, 1)
    scale1 = (1.0 + eps1).reshape(1, 1)
    batch2d = batch.reshape(N // 1000, 1, 1000)

    parts0 = _edge_phase(x, ea0, src, dst)
    h1 = _mlp(x, parts0, W10, b10[None, :], W20, b20[None, :], scale0)
    parts1 = _edge_phase(h1, ea1, src, dst)
    out = _mlp_pool(h1, parts1, W11, b11[None, :], W21, b21[None, :], scale1, batch2d)
    return out
